# baseline (speedup 1.0000x reference)
import functools

import jax
import jax.numpy as jnp
from jax import lax
from jax.experimental import pallas as pl
from jax.experimental.pallas import tpu as pltpu
from jax.experimental.pallas import tpu_sc as plsc

B = 16384
D = 16
NROWS = 1000000
NC = 2
NS = 16
NW = NC * NS
CHUNK = 128
PER_W = B // NW
NCHUNK = PER_W // CHUNK
NGROUP = PER_W // 16


def _sc_body(u_hbm, i_hbm, f_hbm, ub_hbm, uv_hbm, ib_hbm, iv_hbm, par_hbm,
             out_hbm,
             idxu_v, idxi_v, f_v, bu_v, bi_v, xu_v, xi_v, gu_v, gi_v,
             out_v, par_v, sem):
    wid = lax.axis_index("s") * NC + lax.axis_index("c")
    base = wid * PER_W
    row0 = wid * NCHUNK

    pltpu.sync_copy(u_hbm.at[pl.ds(row0, NCHUNK)], idxu_v)
    pltpu.sync_copy(i_hbm.at[pl.ds(row0, NCHUNK)], idxi_v)
    pltpu.sync_copy(f_hbm.at[pl.ds(base, PER_W)], f_v)
    pltpu.sync_copy(par_hbm, par_v)

    def build(g, _):
        j = g // 8
        k = g % 8
        src = pl.ds(k * 16, 16)
        bu = idxu_v[j, src]
        bi = idxi_v[j, src]
        for d in range(D):
            dst = pl.ds((d * NCHUNK + j) * CHUNK + k * 16, 16)
            xu_v[dst] = bu + d * NROWS
            xi_v[dst] = bi + d * NROWS
        return 0

    lax.fori_loop(0, NGROUP, build, 0)

    copies = []
    for j in range(NCHUNK):
        dst = pl.ds(j * CHUNK, CHUNK)
        copies.append(pltpu.async_copy(ub_hbm.at[idxu_v.at[j]], bu_v.at[dst], sem))
        copies.append(pltpu.async_copy(ib_hbm.at[idxi_v.at[j]], bi_v.at[dst], sem))
    for d in range(D):
        for j in range(NCHUNK):
            s = pl.ds((d * NCHUNK + j) * CHUNK, CHUNK)
            copies.append(pltpu.async_copy(uv_hbm.at[xu_v.at[s]], gu_v.at[s], sem))
            copies.append(pltpu.async_copy(iv_hbm.at[xi_v.at[s]], gi_v.at[s], sem))
    for c in copies:
        c.wait()

    w0 = par_v[0]
    w1 = par_v[1]
    cb = par_v[2]

    def group(g, _):
        j = g // 8
        k = g % 8
        off = g * 16
        acc = jnp.zeros((16,), jnp.float32)
        for d in range(D):
            s = pl.ds((d * NCHUNK + j) * CHUNK + k * 16, 16)
            acc = acc + gu_v[s] * gi_v[s]
        fg = f_v[pl.ds(off, 16)]
        out_v[pl.ds(off, 16)] = (cb + bu_v[pl.ds(off, 16)] + bi_v[pl.ds(off, 16)]
                                 + acc + (w0 + w1 * fg) * fg)
        return 0

    lax.fori_loop(0, NGROUP, group, 0)
    pltpu.sync_copy(out_v, out_hbm.at[pl.ds(base, PER_W)])


@jax.jit
def _mfpoly2_sc(u2, i2, f, ub, uvlin, ib, ivlin, params):
    mesh = plsc.VectorSubcoreMesh(core_axis_name="c", subcore_axis_name="s")
    k = functools.partial(
        pl.kernel,
        out_type=jax.ShapeDtypeStruct((B,), jnp.float32),
        mesh=mesh,
        compiler_params=pltpu.CompilerParams(
            needs_layout_passes=False, use_tc_tiling_on_sc=False),
        scratch_types=[
            pltpu.VMEM((NCHUNK, CHUNK), jnp.int32),
            pltpu.VMEM((NCHUNK, CHUNK), jnp.int32),
            pltpu.VMEM((PER_W,), jnp.float32),
            pltpu.VMEM((PER_W,), jnp.float32),
            pltpu.VMEM((PER_W,), jnp.float32),
            pltpu.VMEM((D * PER_W,), jnp.int32),
            pltpu.VMEM((D * PER_W,), jnp.int32),
            pltpu.VMEM((D * PER_W,), jnp.float32),
            pltpu.VMEM((D * PER_W,), jnp.float32),
            pltpu.VMEM((PER_W,), jnp.float32),
            pltpu.VMEM((3, 16), jnp.float32),
            pltpu.SemaphoreType.DMA,
        ],
    )(_sc_body)
    return k(u2, i2, f, ub, uvlin, ib, ivlin, params)


def kernel(u, i, f, user_bias, user_vect, item_bias, item_vect, glob_bias,
           W_frame, b_frame):
    u2 = u.reshape(B // CHUNK, CHUNK).astype(jnp.int32)
    i2 = i.reshape(B // CHUNK, CHUNK).astype(jnp.int32)
    ub = user_bias.reshape(-1)
    ib = item_bias.reshape(-1)
    uvlin = user_vect.T.reshape(-1)
    ivlin = item_vect.T.reshape(-1)
    w = W_frame.reshape(2)
    cb = glob_bias[0] + b_frame[0]
    params = jnp.stack([
        jnp.full((16,), w[0], jnp.float32),
        jnp.full((16,), w[1], jnp.float32),
        jnp.full((16,), cb, jnp.float32),
    ])
    return _mfpoly2_sc(u2, i2, f, ub, uvlin, ib, ivlin, params)


# speedup vs baseline: 18.3119x; 18.3119x over previous
import functools

import jax
import jax.numpy as jnp
from jax import lax
from jax.experimental import pallas as pl
from jax.experimental.pallas import tpu as pltpu
from jax.experimental.pallas import tpu_sc as plsc

B = 16384
D = 16
NROWS = 1000000
M = 999936
NTAIL = NROWS - M
NC = 2
NS = 16
NW = NC * NS
CHUNK = 128
PER_W = B // NW
NCHUNK = PER_W // CHUNK
NGROUP = PER_W // 16
SLAB = 1536
NSLAB = M // SLAB
SLABS_PER_W = -(-NSLAB // NW)


def _worker_id():
    return lax.axis_index("s") * NC + lax.axis_index("c")


def _compact_body(uvt_hbm, ivt_hbm, uv_out, iv_out, bufs, sem_in, sem_out):
    wid = _worker_id()

    def start_in(n, p):
        s = wid + n * NW

        @pl.when(s < NSLAB)
        def _():
            src = pl.ds(s * SLAB, SLAB)
            pltpu.async_copy(uvt_hbm.at[:, src], bufs.at[2 * p], sem_in)
            pltpu.async_copy(ivt_hbm.at[:, src], bufs.at[2 * p + 1], sem_in)

    start_in(0, 0)

    def step(n, _):
        p = n % 2
        s = wid + n * NW

        @pl.when(s < NSLAB)
        def _():
            pltpu.make_async_copy(uvt_hbm.at[:, pl.ds(0, SLAB)],
                                  bufs.at[2 * p], sem_in).wait()
            pltpu.make_async_copy(ivt_hbm.at[:, pl.ds(0, SLAB)],
                                  bufs.at[2 * p + 1], sem_in).wait()
            start_in(n + 1, 1 - p)
            outs = []
            for d in range(D):
                dst = pl.ds(d * M + s * SLAB, SLAB)
                outs.append(pltpu.async_copy(bufs.at[2 * p].at[d],
                                             uv_out.at[dst], sem_out))
                outs.append(pltpu.async_copy(bufs.at[2 * p + 1].at[d],
                                             iv_out.at[dst], sem_out))
            for c in outs:
                c.wait()

        return 0

    lax.fori_loop(0, SLABS_PER_W, step, 0)


def _gather_body(u_hbm, i_hbm, f_hbm, ub_hbm, uv_hbm, ib_hbm, iv_hbm,
                 tails_hbm, par_hbm,
                 out_hbm,
                 idxu_v, idxi_v, f_v, bu_v, bi_v, xu_v, xi_v, gu_v, gi_v,
                 tails_v, out_v, par_v, sem):
    wid = _worker_id()
    base = wid * PER_W
    row0 = wid * NCHUNK

    pltpu.sync_copy(u_hbm.at[pl.ds(row0, NCHUNK)], idxu_v)
    pltpu.sync_copy(i_hbm.at[pl.ds(row0, NCHUNK)], idxi_v)
    pltpu.sync_copy(f_hbm.at[pl.ds(base, PER_W)], f_v)
    pltpu.sync_copy(par_hbm, par_v)
    pltpu.sync_copy(tails_hbm, tails_v)

    def build(g, _):
        j = g // 8
        k = g % 8
        src = pl.ds(k * 16, 16)
        bu = jnp.minimum(idxu_v[j, src], M - 1)
        bi = jnp.minimum(idxi_v[j, src], M - 1)
        for d in range(D):
            dst = pl.ds((d * NCHUNK + j) * CHUNK + k * 16, 16)
            xu_v[dst] = bu + d * M
            xi_v[dst] = bi + d * M
        return 0

    lax.fori_loop(0, NGROUP, build, 0)

    copies = []
    for j in range(NCHUNK):
        dst = pl.ds(j * CHUNK, CHUNK)
        copies.append(pltpu.async_copy(ub_hbm.at[idxu_v.at[j]], bu_v.at[dst], sem))
        copies.append(pltpu.async_copy(ib_hbm.at[idxi_v.at[j]], bi_v.at[dst], sem))
    for d in range(D):
        for j in range(NCHUNK):
            s = pl.ds((d * NCHUNK + j) * CHUNK, CHUNK)
            copies.append(pltpu.async_copy(uv_hbm.at[xu_v.at[s]], gu_v.at[s], sem))
            copies.append(pltpu.async_copy(iv_hbm.at[xi_v.at[s]], gi_v.at[s], sem))
    for c in copies:
        c.wait()

    w0 = par_v[0]
    w1 = par_v[1]
    cb = par_v[2]

    def group(g, _):
        j = g // 8
        k = g % 8
        off = g * 16
        src = pl.ds(k * 16, 16)
        iu = idxu_v[j, src]
        ii = idxi_v[j, src]
        mu = iu >= M
        mi = ii >= M
        tu = jnp.clip(iu - M, 0, NTAIL - 1)
        ti = jnp.clip(ii - M, 0, NTAIL - 1) + NTAIL
        acc = jnp.zeros((16,), jnp.float32)
        for d in range(D):
            s = pl.ds((d * NCHUNK + j) * CHUNK + k * 16, 16)
            cd = jnp.full((16,), d, dtype=jnp.int32)
            uval = jnp.where(mu, plsc.load_gather(tails_v, [tu, cd]), gu_v[s])
            ival = jnp.where(mi, plsc.load_gather(tails_v, [ti, cd]), gi_v[s])
            acc = acc + uval * ival
        fg = f_v[pl.ds(off, 16)]
        out_v[pl.ds(off, 16)] = (cb + bu_v[pl.ds(off, 16)] + bi_v[pl.ds(off, 16)]
                                 + acc + (w0 + w1 * fg) * fg)
        return 0

    lax.fori_loop(0, NGROUP, group, 0)
    pltpu.sync_copy(out_v, out_hbm.at[pl.ds(base, PER_W)])


@jax.jit
def _mfpoly2_sc(u2, i2, f, ub, uvt, ib, ivt, tails, params):
    mesh = plsc.VectorSubcoreMesh(core_axis_name="c", subcore_axis_name="s")

    compact = functools.partial(
        pl.kernel,
        out_type=(jax.ShapeDtypeStruct((D * M,), jnp.float32),
                  jax.ShapeDtypeStruct((D * M,), jnp.float32)),
        mesh=mesh,
        compiler_params=pltpu.CompilerParams(
            needs_layout_passes=False, use_tc_tiling_on_sc=True),
        scratch_types=[
            pltpu.VMEM((4, D, SLAB), jnp.float32),
            pltpu.SemaphoreType.DMA,
            pltpu.SemaphoreType.DMA,
        ],
    )(_compact_body)
    uv_lin, iv_lin = compact(uvt, ivt)

    gather = functools.partial(
        pl.kernel,
        out_type=jax.ShapeDtypeStruct((B,), jnp.float32),
        mesh=mesh,
        compiler_params=pltpu.CompilerParams(
            needs_layout_passes=False, use_tc_tiling_on_sc=False),
        scratch_types=[
            pltpu.VMEM((NCHUNK, CHUNK), jnp.int32),
            pltpu.VMEM((NCHUNK, CHUNK), jnp.int32),
            pltpu.VMEM((PER_W,), jnp.float32),
            pltpu.VMEM((PER_W,), jnp.float32),
            pltpu.VMEM((PER_W,), jnp.float32),
            pltpu.VMEM((D * PER_W,), jnp.int32),
            pltpu.VMEM((D * PER_W,), jnp.int32),
            pltpu.VMEM((D * PER_W,), jnp.float32),
            pltpu.VMEM((D * PER_W,), jnp.float32),
            pltpu.VMEM((2 * NTAIL, D), jnp.float32),
            pltpu.VMEM((PER_W,), jnp.float32),
            pltpu.VMEM((3, 16), jnp.float32),
            pltpu.SemaphoreType.DMA,
        ],
    )(_gather_body)
    return gather(u2, i2, f, ub, uv_lin, ib, iv_lin, tails, params)


def kernel(u, i, f, user_bias, user_vect, item_bias, item_vect, glob_bias,
           W_frame, b_frame):
    u2 = u.reshape(B // CHUNK, CHUNK).astype(jnp.int32)
    i2 = i.reshape(B // CHUNK, CHUNK).astype(jnp.int32)
    ub = user_bias.reshape(-1)
    ib = item_bias.reshape(-1)
    uvt = user_vect.T
    ivt = item_vect.T
    tails = jnp.concatenate([user_vect[M:], item_vect[M:]], axis=0)
    w = W_frame.reshape(2)
    cb = glob_bias[0] + b_frame[0]
    params = jnp.stack([
        jnp.full((16,), w[0], jnp.float32),
        jnp.full((16,), w[1], jnp.float32),
        jnp.full((16,), cb, jnp.float32),
    ])
    return _mfpoly2_sc(u2, i2, f, ub, uvt, ib, ivt, tails, params)
